# baseline (speedup 1.0000x reference)
import functools

import jax
import jax.numpy as jnp
from jax import lax
from jax.experimental import pallas as pl
from jax.experimental.pallas import tpu as pltpu
from jax.experimental.pallas import tpu_sc as plsc

_LANES = 16


def _make_sc_kernel(B, S, D, nw, ch_rows):
    chunk = ch_rows * D
    sw = S // nw
    n_chunks = sw // ch_rows
    mesh = plsc.VectorSubcoreMesh(core_axis_name="c", subcore_axis_name="s")

    @functools.partial(
        pl.kernel,
        mesh=mesh,
        out_type=jax.ShapeDtypeStruct((B * S * D,), jnp.float32),
        scratch_types=[
            pltpu.VMEM((chunk,), jnp.float32),
            pltpu.VMEM((chunk,), jnp.float32),
        ],
    )
    def sc_kernel(x_hbm, pe_hbm, out_hbm, pebuf, xbuf):
        nc = mesh.shape["c"]
        wid = lax.axis_index("s") * nc + lax.axis_index("c")
        base_s = wid * (sw * D)

        def chunk_body(c, _):
            pe_off = base_s + c * chunk
            pltpu.sync_copy(pe_hbm.at[pl.ds(pe_off, chunk)], pebuf)
            for b in range(B):
                x_off = b * (S * D) + pe_off
                pltpu.sync_copy(x_hbm.at[pl.ds(x_off, chunk)], xbuf)

                def add_body(i, _):
                    sl = pl.ds(i * _LANES, _LANES)
                    xbuf[sl] = xbuf[sl] + pebuf[sl]
                    return 0

                lax.fori_loop(0, chunk // _LANES, add_body, 0)
                pltpu.sync_copy(xbuf, out_hbm.at[pl.ds(x_off, chunk)])
            return 0

        lax.fori_loop(0, n_chunks, chunk_body, 0)

    return sc_kernel


def kernel(x, pe):
    B, S, D = x.shape
    info = plsc.get_sparse_core_info()
    nw = info.num_cores * info.num_subcores
    sc = _make_sc_kernel(B, S, D, nw, ch_rows=16)
    out = sc(x.reshape(-1), pe[:S].reshape(-1))
    return out.reshape(B, S, D)


# speedup vs baseline: 1.8434x; 1.8434x over previous
import functools

import jax
import jax.numpy as jnp
from jax import lax
from jax.experimental import pallas as pl
from jax.experimental.pallas import tpu as pltpu
from jax.experimental.pallas import tpu_sc as plsc

_LANES = 16
_UNROLL = 8


def _make_sc_kernel(B, S, D, nw, ch_rows):
    chunk = ch_rows * D
    sw = S // nw
    n_chunks = sw // ch_rows
    n_items = n_chunks * B
    mesh = plsc.VectorSubcoreMesh(core_axis_name="c", subcore_axis_name="s")

    @functools.partial(
        pl.kernel,
        mesh=mesh,
        out_type=jax.ShapeDtypeStruct((B * S * D,), jnp.float32),
        scratch_types=[
            pltpu.VMEM((chunk,), jnp.float32),
            pltpu.VMEM((chunk,), jnp.float32),
            pltpu.VMEM((chunk,), jnp.float32),
            pltpu.VMEM((chunk,), jnp.float32),
            pltpu.VMEM((chunk,), jnp.float32),
            pltpu.VMEM((chunk,), jnp.float32),
            pltpu.SemaphoreType.DMA,
            pltpu.SemaphoreType.DMA,
            pltpu.SemaphoreType.DMA,
            pltpu.SemaphoreType.DMA,
            pltpu.SemaphoreType.DMA,
            pltpu.SemaphoreType.DMA,
        ],
    )
    def sc_kernel(x_hbm, pe_hbm, out_hbm,
                  xb0, xb1, ob0, ob1, pb0, pb1,
                  sx0, sx1, so0, so1, spe0, spe1):
        nc = mesh.shape["c"]
        wid = lax.axis_index("s") * nc + lax.axis_index("c")
        base = wid * (sw * D)

        xb, ob, pb = (xb0, xb1), (ob0, ob1), (pb0, pb1)
        sx, so, spe = (sx0, sx1), (so0, so1), (spe0, spe1)

        def x_off(k):
            return (k % B) * (S * D) + base + (k // B) * chunk

        def start_x(k):
            return pltpu.async_copy(
                x_hbm.at[pl.ds(x_off(k), chunk)], xb[k % 2], sx[k % 2])

        def start_pe(c):
            return pltpu.async_copy(
                pe_hbm.at[pl.ds(base + c * chunk, chunk)], pb[c % 2], spe[c % 2])

        def start_out(k):
            return pltpu.async_copy(
                ob[k % 2], out_hbm.at[pl.ds(x_off(k), chunk)], so[k % 2])

        x_dma = {0: start_x(0)}
        pe_dma = {0: start_pe(0)}
        out_dma = {}

        for k in range(n_items):
            if k + 1 < n_items:
                if (k + 1) % B == 0:
                    pe_dma[k // B + 1] = start_pe(k // B + 1)
                x_dma[k + 1] = start_x(k + 1)
            x_dma[k].wait()
            if k % B == 0:
                pe_dma[k // B].wait()
            if k >= 2:
                out_dma[k - 2].wait()

            xbuf, obuf, pbuf = xb[k % 2], ob[k % 2], pb[(k // B) % 2]

            def add_body(t, _, xbuf=xbuf, obuf=obuf, pbuf=pbuf):
                b0 = t * (_LANES * _UNROLL)
                for u in range(_UNROLL):
                    sl = pl.ds(b0 + u * _LANES, _LANES)
                    obuf[sl] = xbuf[sl] + pbuf[sl]
                return 0

            lax.fori_loop(0, chunk // (_LANES * _UNROLL), add_body, 0)
            out_dma[k] = start_out(k)

        out_dma[n_items - 2].wait()
        out_dma[n_items - 1].wait()

    return sc_kernel


def kernel(x, pe):
    B, S, D = x.shape
    info = plsc.get_sparse_core_info()
    nw = info.num_cores * info.num_subcores
    sc = _make_sc_kernel(B, S, D, nw, ch_rows=16)
    out = sc(x.reshape(-1), pe[:S].reshape(-1))
    return out.reshape(B, S, D)


# speedup vs baseline: 1.8490x; 1.0030x over previous
import functools

import jax
import jax.numpy as jnp
from jax import lax
from jax.experimental import pallas as pl
from jax.experimental.pallas import tpu as pltpu
from jax.experimental.pallas import tpu_sc as plsc

_LANES = 16
_UNROLL = 8


def _make_sc_kernel(B, S, D, nw, ch_rows):
    chunk = ch_rows * D
    sw = S // nw
    n_chunks = sw // ch_rows
    n_items = n_chunks * B
    mesh = plsc.VectorSubcoreMesh(core_axis_name="c", subcore_axis_name="s")

    @functools.partial(
        pl.kernel,
        mesh=mesh,
        out_type=jax.ShapeDtypeStruct((B * S * D,), jnp.float32),
        scratch_types=[
            pltpu.VMEM((chunk,), jnp.float32),
            pltpu.VMEM((chunk,), jnp.float32),
            pltpu.VMEM((chunk,), jnp.float32),
            pltpu.VMEM((chunk,), jnp.float32),
            pltpu.VMEM((chunk,), jnp.float32),
            pltpu.VMEM((chunk,), jnp.float32),
            pltpu.SemaphoreType.DMA,
            pltpu.SemaphoreType.DMA,
            pltpu.SemaphoreType.DMA,
            pltpu.SemaphoreType.DMA,
            pltpu.SemaphoreType.DMA,
            pltpu.SemaphoreType.DMA,
        ],
    )
    def sc_kernel(x_hbm, pe_hbm, out_hbm,
                  xb0, xb1, ob0, ob1, pb0, pb1,
                  sx0, sx1, so0, so1, spe0, spe1):
        nc = mesh.shape["c"]
        wid = lax.axis_index("s") * nc + lax.axis_index("c")
        base = wid * (sw * D)

        xb, ob, pb = (xb0, xb1), (ob0, ob1), (pb0, pb1)
        sx, so, spe = (sx0, sx1), (so0, so1), (spe0, spe1)

        def x_off(k):
            return (k % B) * (S * D) + base + (k // B) * chunk

        def start_x(k):
            return pltpu.async_copy(
                x_hbm.at[pl.ds(x_off(k), chunk)], xb[k % 2], sx[k % 2])

        def start_pe(c):
            return pltpu.async_copy(
                pe_hbm.at[pl.ds(base + c * chunk, chunk)], pb[c % 2], spe[c % 2])

        def start_out(k):
            return pltpu.async_copy(
                ob[k % 2], out_hbm.at[pl.ds(x_off(k), chunk)], so[k % 2])

        x_dma = {0: start_x(0)}
        pe_dma = {0: start_pe(0)}
        out_dma = {}

        for k in range(n_items):
            if k + 1 < n_items:
                if (k + 1) % B == 0:
                    pe_dma[k // B + 1] = start_pe(k // B + 1)
                x_dma[k + 1] = start_x(k + 1)
            x_dma[k].wait()
            if k % B == 0:
                pe_dma[k // B].wait()
            if k >= 2:
                out_dma[k - 2].wait()

            xbuf, obuf, pbuf = xb[k % 2], ob[k % 2], pb[(k // B) % 2]

            @plsc.parallel_loop(0, chunk, step=_LANES, unroll=_UNROLL)
            def add_body(i, xbuf=xbuf, obuf=obuf, pbuf=pbuf):
                sl = pl.ds(i, _LANES)
                obuf[sl] = xbuf[sl] + pbuf[sl]

            out_dma[k] = start_out(k)

        out_dma[n_items - 2].wait()
        out_dma[n_items - 1].wait()

    return sc_kernel


def kernel(x, pe):
    B, S, D = x.shape
    info = plsc.get_sparse_core_info()
    nw = info.num_cores * info.num_subcores
    sc = _make_sc_kernel(B, S, D, nw, ch_rows=16)
    out = sc(x.reshape(-1), pe[:S].reshape(-1))
    return out.reshape(B, S, D)


# speedup vs baseline: 7.1020x; 3.8410x over previous
import jax
import jax.numpy as jnp
from jax.experimental import pallas as pl


def _make_add_kernel(S, Sb):
    def add_kernel(x_ref, pe_ref, o_ref):
        i = pl.program_id(0)
        s0 = (i * Sb) % S
        o_ref[...] = x_ref[...] + pe_ref[pl.ds(s0, Sb), :]

    return add_kernel


def kernel(x, pe):
    B, S, D = x.shape
    Sb = 512
    xf = x.reshape(B * S, D)
    return pl.pallas_call(
        _make_add_kernel(S, Sb),
        grid=(B * S // Sb,),
        in_specs=[
            pl.BlockSpec((Sb, D), lambda i: (i, 0)),
            pl.BlockSpec((S, D), lambda i: (0, 0)),
        ],
        out_specs=pl.BlockSpec((Sb, D), lambda i: (i, 0)),
        out_shape=jax.ShapeDtypeStruct((B * S, D), x.dtype),
    )(xf, pe[:S]).reshape(B, S, D)
